# baseline (speedup 1.0000x reference)
import functools

import jax
import jax.numpy as jnp
from jax import lax
from jax.experimental import pallas as pl
from jax.experimental.pallas import tpu as pltpu
from jax.experimental.pallas import tpu_sc as plsc

N = 10000
E = 320000
H = 128
NL = 3
DOUT = 45
EPS = 1e-5

NC = 2
NS = 16
NW = NC * NS
CH = 128
G = 24
EPG = G * CH
NG0 = 7
NG1 = 0
TG = NS * (NG0 + NG1)
EPAD = TG * EPG
RPT = 632
NPAD = NS * RPT


_SC_MESH = plsc.VectorSubcoreMesh(core_axis_name="c", subcore_axis_name="s")


@functools.partial(
    pl.kernel,
    out_type=jax.ShapeDtypeStruct((NW, RPT, H), jnp.float32),
    mesh=_SC_MESH,
    scratch_types=[
        pltpu.VMEM((G, CH), jnp.int32),
        pltpu.VMEM((G, CH), jnp.int32),
        pltpu.VMEM((CH, H), jnp.float32),
        pltpu.VMEM((CH, H), jnp.float32),
        pltpu.VMEM_SHARED((NPAD, H), jnp.float32),
        pltpu.SemaphoreType.DMA,
    ],
)
def _spmm_h(table_hbm, src_hbm, dst_hbm, zeros_hbm, out_hbm,
            src_v, dst_v, rows_a, rows_b, acc, sem):
    c = lax.axis_index("c")
    s = lax.axis_index("s")
    wid = c * NS + s
    row0 = s * RPT
    pltpu.sync_copy(zeros_hbm.at[pl.ds(row0, RPT)], acc.at[pl.ds(row0, RPT)])
    plsc.subcore_barrier()

    def step(i, carry):
        j = 2 * i
        pltpu.make_async_copy(table_hbm.at[src_v.at[j]], rows_a, sem).wait()
        pltpu.async_copy(table_hbm.at[src_v.at[j + 1]], rows_b, sem)
        pltpu.sync_copy(rows_a, acc.at[dst_v.at[j]], add=True)
        pltpu.make_async_copy(table_hbm.at[src_v.at[j + 1]], rows_b, sem).wait()

        @pl.when(j + 2 < G)
        def _():
            pltpu.async_copy(table_hbm.at[src_v.at[j + 2]], rows_a, sem)

        pltpu.sync_copy(rows_b, acc.at[dst_v.at[j + 1]], add=True)
        return carry

    def do_groups(gbase, n):
        for g in range(n):
            pltpu.sync_copy(src_hbm.at[gbase + g], src_v)
            pltpu.sync_copy(dst_hbm.at[gbase + g], dst_v)
            pltpu.async_copy(table_hbm.at[src_v.at[0]], rows_a, sem)
            lax.fori_loop(0, G // 2, step, 0)

    @pl.when(c == 0)
    def _():
        do_groups(s * NG0, NG0)

    plsc.subcore_barrier()
    pltpu.sync_copy(acc.at[pl.ds(row0, RPT)], out_hbm.at[wid])


@functools.partial(
    pl.kernel,
    out_type=jax.ShapeDtypeStruct((NW, RPT, H), jnp.float32),
    mesh=_SC_MESH,
    scratch_types=[
        pltpu.VMEM((G, CH), jnp.int32),
        pltpu.VMEM((CH, H), jnp.float32),
        pltpu.VMEM_SHARED((NPAD, H), jnp.float32),
    ],
)
def _spmm_cnt(ones_hbm, dst_hbm, zeros_hbm, out_hbm, dst_v, ones_v, acc):
    c = lax.axis_index("c")
    s = lax.axis_index("s")
    wid = c * NS + s
    pltpu.sync_copy(ones_hbm, ones_v)
    row0 = s * RPT
    pltpu.sync_copy(zeros_hbm.at[pl.ds(row0, RPT)], acc.at[pl.ds(row0, RPT)])
    plsc.subcore_barrier()

    ngroups = jnp.where(wid < NS, 4, 3)
    gbase = jnp.where(wid < NS, wid * 4, 4 * NS + (wid - NS) * 3)

    def step(j, carry):
        pltpu.sync_copy(ones_v, acc.at[dst_v.at[j]], add=True)
        return carry

    def group(g, carry):
        pltpu.sync_copy(dst_hbm.at[gbase + g], dst_v)
        lax.fori_loop(0, G, step, 0)
        return carry

    lax.fori_loop(0, ngroups, group, 0)

    plsc.subcore_barrier()
    pltpu.sync_copy(acc.at[pl.ds(row0, RPT)], out_hbm.at[wid])


def _lnorm(x, g, b):
    m = jnp.mean(x, axis=-1, keepdims=True)
    v = jnp.var(x, axis=-1, keepdims=True)
    return (x - m) / jnp.sqrt(v + EPS) * g + b


def _leaky(x):
    return jnp.where(x > 0, x, 0.1 * x)


def _gelu(x):
    return 0.5 * x * (1.0 + lax.erf(x * (2.0 ** -0.5)))


def _encoder_body(x_ref, w1, b1, g1, be1, w2, b2, g2, be2, o_ref):
    h = jnp.dot(x_ref[...], w1[...], preferred_element_type=jnp.float32) + b1[...]
    h = _lnorm(h, g1[...], be1[...])
    h = _leaky(h)
    h = jnp.dot(h, w2[...], preferred_element_type=jnp.float32) + b2[...]
    o_ref[...] = _lnorm(h, g2[...], be2[...])


def _layer_body(part, cntp, h_ref, skw, skb, wl, bl, wr, g, b, o_ref):
    h = h_ref[...]
    cnt = (cntp[0] + cntp[1])[:N, :1]
    cnt = jnp.maximum(cnt, 1.0)
    agg = (part[0] + part[1])[:N] / cnt
    out = (jnp.dot(agg, wl[...], preferred_element_type=jnp.float32) + bl[...]
           + jnp.dot(h, wr[...], preferred_element_type=jnp.float32))
    nrm = jnp.sqrt(jnp.sum(out * out, axis=-1, keepdims=True))
    out = out / jnp.maximum(nrm, 1e-12)
    m = jnp.mean(out, axis=0, keepdims=True)
    v = jnp.var(out, axis=0, keepdims=True)
    out = (out - m) / jnp.sqrt(v + EPS) * g[...] + b[...]
    res = jnp.dot(h, skw[...], preferred_element_type=jnp.float32) + skb[...]
    o_ref[...] = _leaky(out) + res


def _head_body(h_ref, w1, b1, g, be, w2, b2, w3, b3, o_ref):
    y = jnp.dot(h_ref[...], w1[...], preferred_element_type=jnp.float32) + b1[...]
    y = _lnorm(y, g[...], be[...])
    y = _gelu(y)
    y = jnp.dot(y, w2[...], preferred_element_type=jnp.float32) + b2[...]
    y = _gelu(y)
    y = jnp.dot(y, w3[...], preferred_element_type=jnp.float32) + b3[...]
    o_ref[...] = jax.nn.sigmoid(y)


def kernel(x, edge_index, params):
    p = params
    src = edge_index[0].astype(jnp.int32)
    dst = edge_index[1].astype(jnp.int32)
    pad = EPAD - E
    src = jnp.concatenate([src, jnp.zeros((pad,), jnp.int32)]).reshape(TG, G, CH)
    dst = jnp.concatenate([dst, jnp.full((pad,), N, jnp.int32)]).reshape(TG, G, CH)

    h = pl.pallas_call(
        _encoder_body,
        out_shape=jax.ShapeDtypeStruct((N, H), jnp.float32),
    )(x, p['enc_W1'], p['enc_b1'], p['enc_g1'], p['enc_be1'],
      p['enc_W2'], p['enc_b2'], p['enc_g2'], p['enc_be2'])

    ones_ch = jnp.ones((CH, H), jnp.float32)
    zh = jnp.zeros((NPAD, H), jnp.float32)

    cntp = _spmm_cnt(ones_ch, dst, zh).reshape(NC, NPAD, H)

    for l in range(NL):
        part = _spmm_h(h, src, dst, zh).reshape(NC, NPAD, H)
        h = pl.pallas_call(
            _layer_body,
            out_shape=jax.ShapeDtypeStruct((N, H), jnp.float32),
        )(part, cntp, h, p['skip_W'], p['skip_b'],
          p[f'sage_Wl{l}'], p[f'sage_bl{l}'], p[f'sage_Wr{l}'],
          p[f'bn_g{l}'], p[f'bn_b{l}'])

    y = pl.pallas_call(
        _head_body,
        out_shape=jax.ShapeDtypeStruct((N, DOUT), jnp.float32),
    )(h, p['head_W1'], p['head_b1'], p['head_g'], p['head_be'],
      p['head_W2'], p['head_b2'], p['head_W3'], p['head_b3'])
    return y


# speedup vs baseline: 2.0626x; 2.0626x over previous
import functools

import jax
import jax.numpy as jnp
from jax import lax
from jax.experimental import pallas as pl
from jax.experimental.pallas import tpu as pltpu
from jax.experimental.pallas import tpu_sc as plsc

N = 10000
E = 320000
H = 128
NL = 3
DOUT = 45
EPS = 1e-5

NC = 2
NS = 16
NW = NC * NS
CH = 128
G = 40
EPG = G * CH
NG0 = 4
NG1 = 0
TG = NS * (NG0 + NG1)
EPAD = TG * EPG
RPT = 632
NPAD = NS * RPT


_SC_MESH = plsc.VectorSubcoreMesh(core_axis_name="c", subcore_axis_name="s")


@functools.partial(
    pl.kernel,
    out_type=jax.ShapeDtypeStruct((NW, RPT, H), jnp.float32),
    mesh=_SC_MESH,
    scratch_types=[
        pltpu.VMEM((G, CH), jnp.int32),
        pltpu.VMEM((G, CH), jnp.int32),
        pltpu.VMEM((CH, H), jnp.float32),
        pltpu.VMEM((CH, H), jnp.float32),
        pltpu.VMEM_SHARED((NPAD, H), jnp.float32),
        pltpu.SemaphoreType.DMA,
    ],
)
def _spmm_h(table_hbm, src_hbm, dst_hbm, zeros_hbm, out_hbm,
            src_v, dst_v, rows_a, rows_b, acc, sem):
    c = lax.axis_index("c")
    s = lax.axis_index("s")
    wid = c * NS + s
    row0 = s * RPT
    pltpu.sync_copy(zeros_hbm.at[pl.ds(row0, RPT)], acc.at[pl.ds(row0, RPT)])
    plsc.subcore_barrier()

    def step(i, carry):
        j = 2 * i
        pltpu.make_async_copy(table_hbm.at[src_v.at[j]], rows_a, sem).wait()
        pltpu.async_copy(table_hbm.at[src_v.at[j + 1]], rows_b, sem)
        pltpu.sync_copy(rows_a, acc.at[dst_v.at[j]], add=True)
        pltpu.make_async_copy(table_hbm.at[src_v.at[j + 1]], rows_b, sem).wait()

        @pl.when(j + 2 < G)
        def _():
            pltpu.async_copy(table_hbm.at[src_v.at[j + 2]], rows_a, sem)

        pltpu.sync_copy(rows_b, acc.at[dst_v.at[j + 1]], add=True)
        return carry

    def do_groups(gbase, n):
        for g in range(n):
            pltpu.sync_copy(src_hbm.at[gbase + g], src_v)
            pltpu.sync_copy(dst_hbm.at[gbase + g], dst_v)
            pltpu.async_copy(table_hbm.at[src_v.at[0]], rows_a, sem)
            lax.fori_loop(0, G // 2, step, 0)

    @pl.when(c == 0)
    def _():
        do_groups(s * NG0, NG0)

    plsc.subcore_barrier()
    pltpu.sync_copy(acc.at[pl.ds(row0, RPT)], out_hbm.at[wid])


@functools.partial(
    pl.kernel,
    out_type=jax.ShapeDtypeStruct((NW, RPT, H), jnp.float32),
    mesh=_SC_MESH,
    scratch_types=[
        pltpu.VMEM((G, CH), jnp.int32),
        pltpu.VMEM((CH, H), jnp.float32),
        pltpu.VMEM_SHARED((NPAD, H), jnp.float32),
    ],
)
def _spmm_cnt(ones_hbm, dst_hbm, zeros_hbm, out_hbm, dst_v, ones_v, acc):
    c = lax.axis_index("c")
    s = lax.axis_index("s")
    wid = c * NS + s
    pltpu.sync_copy(ones_hbm, ones_v)
    row0 = s * RPT
    pltpu.sync_copy(zeros_hbm.at[pl.ds(row0, RPT)], acc.at[pl.ds(row0, RPT)])
    plsc.subcore_barrier()

    ngroups = TG // NW

    def step(j, carry):
        pltpu.sync_copy(ones_v, acc.at[dst_v.at[j]], add=True)
        return carry

    for g in range(ngroups):
        pltpu.sync_copy(dst_hbm.at[wid * ngroups + g], dst_v)
        lax.fori_loop(0, G, step, 0)

    plsc.subcore_barrier()
    pltpu.sync_copy(acc.at[pl.ds(row0, RPT)], out_hbm.at[wid])


def _lnorm(x, g, b):
    m = jnp.mean(x, axis=-1, keepdims=True)
    v = jnp.var(x, axis=-1, keepdims=True)
    return (x - m) / jnp.sqrt(v + EPS) * g + b


def _leaky(x):
    return jnp.where(x > 0, x, 0.1 * x)


def _gelu(x):
    return 0.5 * x * (1.0 + lax.erf(x * (2.0 ** -0.5)))


def _encoder_body(x_ref, w1, b1, g1, be1, w2, b2, g2, be2, o_ref):
    h = jnp.dot(x_ref[...], w1[...], preferred_element_type=jnp.float32) + b1[...]
    h = _lnorm(h, g1[...], be1[...])
    h = _leaky(h)
    h = jnp.dot(h, w2[...], preferred_element_type=jnp.float32) + b2[...]
    o_ref[...] = _lnorm(h, g2[...], be2[...])


def _layer_body(part, cntp, h_ref, skw, skb, wl, bl, wr, g, b, o_ref):
    h = h_ref[...]
    cnt = (cntp[0] + cntp[1])[:N, :1]
    cnt = jnp.maximum(cnt, 1.0)
    agg = (part[0] + part[1])[:N] / cnt
    out = (jnp.dot(agg, wl[...], preferred_element_type=jnp.float32) + bl[...]
           + jnp.dot(h, wr[...], preferred_element_type=jnp.float32))
    nrm = jnp.sqrt(jnp.sum(out * out, axis=-1, keepdims=True))
    out = out / jnp.maximum(nrm, 1e-12)
    m = jnp.mean(out, axis=0, keepdims=True)
    v = jnp.var(out, axis=0, keepdims=True)
    out = (out - m) / jnp.sqrt(v + EPS) * g[...] + b[...]
    res = jnp.dot(h, skw[...], preferred_element_type=jnp.float32) + skb[...]
    o_ref[...] = _leaky(out) + res


def _head_body(h_ref, w1, b1, g, be, w2, b2, w3, b3, o_ref):
    y = jnp.dot(h_ref[...], w1[...], preferred_element_type=jnp.float32) + b1[...]
    y = _lnorm(y, g[...], be[...])
    y = _gelu(y)
    y = jnp.dot(y, w2[...], preferred_element_type=jnp.float32) + b2[...]
    y = _gelu(y)
    y = jnp.dot(y, w3[...], preferred_element_type=jnp.float32) + b3[...]
    o_ref[...] = jax.nn.sigmoid(y)


def kernel(x, edge_index, params):
    p = params
    src = edge_index[0].astype(jnp.int32)
    dst = edge_index[1].astype(jnp.int32)
    pad = EPAD - E
    src = jnp.concatenate([src, jnp.zeros((pad,), jnp.int32)]).reshape(TG, G, CH)
    dst = jnp.concatenate([dst, jnp.full((pad,), N, jnp.int32)]).reshape(TG, G, CH)

    h = pl.pallas_call(
        _encoder_body,
        out_shape=jax.ShapeDtypeStruct((N, H), jnp.float32),
    )(x, p['enc_W1'], p['enc_b1'], p['enc_g1'], p['enc_be1'],
      p['enc_W2'], p['enc_b2'], p['enc_g2'], p['enc_be2'])

    ones_ch = jnp.ones((CH, H), jnp.float32)
    zh = jnp.zeros((NPAD, H), jnp.float32)

    cntp = _spmm_cnt(ones_ch, dst, zh).reshape(NC, NPAD, H)

    for l in range(NL):
        part = _spmm_h(h, src, dst, zh).reshape(NC, NPAD, H)
        h = pl.pallas_call(
            _layer_body,
            out_shape=jax.ShapeDtypeStruct((N, H), jnp.float32),
        )(part, cntp, h, p['skip_W'], p['skip_b'],
          p[f'sage_Wl{l}'], p[f'sage_bl{l}'], p[f'sage_Wr{l}'],
          p[f'bn_g{l}'], p[f'bn_b{l}'])

    y = pl.pallas_call(
        _head_body,
        out_shape=jax.ShapeDtypeStruct((N, DOUT), jnp.float32),
    )(h, p['head_W1'], p['head_b1'], p['head_g'], p['head_be'],
      p['head_W2'], p['head_b2'], p['head_W3'], p['head_b3'])
    return y


# speedup vs baseline: 2.8209x; 1.3677x over previous
import functools

import jax
import jax.numpy as jnp
from jax import lax
from jax.experimental import pallas as pl
from jax.experimental.pallas import tpu as pltpu
from jax.experimental.pallas import tpu_sc as plsc

N = 10000
E = 320000
H = 128
NL = 3
DOUT = 45
EPS = 1e-5

NC = 2
NS = 16
NW = NC * NS
CH = 128
G = 40
EPG = G * CH
NG0 = 3
NG1 = 1
TG = NS * (NG0 + NG1)
EPAD = TG * EPG
RPT = 632
NPAD = NS * RPT


_SC_MESH = plsc.VectorSubcoreMesh(core_axis_name="c", subcore_axis_name="s")


@functools.partial(
    pl.kernel,
    out_type=jax.ShapeDtypeStruct((NW, RPT, H), jnp.float32),
    mesh=_SC_MESH,
    scratch_types=[
        pltpu.VMEM((G, CH), jnp.int32),
        pltpu.VMEM((G, CH), jnp.int32),
        pltpu.VMEM((CH, H), jnp.float32),
        pltpu.VMEM((CH, H), jnp.float32),
        pltpu.VMEM_SHARED((NPAD, H), jnp.float32),
        pltpu.SemaphoreType.DMA,
    ],
)
def _spmm_h(table_hbm, src_hbm, dst_hbm, zeros_hbm, out_hbm,
            src_v, dst_v, rows_a, rows_b, acc, sem):
    c = lax.axis_index("c")
    s = lax.axis_index("s")
    wid = c * NS + s
    row0 = s * RPT
    pltpu.sync_copy(zeros_hbm.at[pl.ds(row0, RPT)], acc.at[pl.ds(row0, RPT)])
    plsc.subcore_barrier()

    def step(i, carry):
        j = 2 * i
        pltpu.make_async_copy(table_hbm.at[src_v.at[j]], rows_a, sem).wait()
        pltpu.async_copy(table_hbm.at[src_v.at[j + 1]], rows_b, sem)
        pltpu.sync_copy(rows_a, acc.at[dst_v.at[j]], add=True)
        pltpu.make_async_copy(table_hbm.at[src_v.at[j + 1]], rows_b, sem).wait()

        @pl.when(j + 2 < G)
        def _():
            pltpu.async_copy(table_hbm.at[src_v.at[j + 2]], rows_a, sem)

        pltpu.sync_copy(rows_b, acc.at[dst_v.at[j + 1]], add=True)
        return carry

    def do_groups(gbase, n):
        for g in range(n):
            pltpu.sync_copy(src_hbm.at[gbase + g], src_v)
            pltpu.sync_copy(dst_hbm.at[gbase + g], dst_v)
            pltpu.async_copy(table_hbm.at[src_v.at[0]], rows_a, sem)
            lax.fori_loop(0, G // 2, step, 0)

    @pl.when(c == 0)
    def _():
        do_groups(s * NG0, NG0)

    if NG1:
        @pl.when(c == 1)
        def _():
            do_groups(NS * NG0 + s, NG1)

    plsc.subcore_barrier()
    pltpu.sync_copy(acc.at[pl.ds(row0, RPT)], out_hbm.at[wid])


@functools.partial(
    pl.kernel,
    out_type=jax.ShapeDtypeStruct((NW, RPT, H), jnp.float32),
    mesh=_SC_MESH,
    scratch_types=[
        pltpu.VMEM((G, CH), jnp.int32),
        pltpu.VMEM((CH, H), jnp.float32),
        pltpu.VMEM_SHARED((NPAD, H), jnp.float32),
    ],
)
def _spmm_cnt(ones_hbm, dst_hbm, zeros_hbm, out_hbm, dst_v, ones_v, acc):
    c = lax.axis_index("c")
    s = lax.axis_index("s")
    wid = c * NS + s
    pltpu.sync_copy(ones_hbm, ones_v)
    row0 = s * RPT
    pltpu.sync_copy(zeros_hbm.at[pl.ds(row0, RPT)], acc.at[pl.ds(row0, RPT)])
    plsc.subcore_barrier()

    ngroups = TG // NW

    def step(j, carry):
        pltpu.sync_copy(ones_v, acc.at[dst_v.at[j]], add=True)
        return carry

    for g in range(ngroups):
        pltpu.sync_copy(dst_hbm.at[wid * ngroups + g], dst_v)
        lax.fori_loop(0, G, step, 0)

    plsc.subcore_barrier()
    pltpu.sync_copy(acc.at[pl.ds(row0, RPT)], out_hbm.at[wid])


def _lnorm(x, g, b):
    m = jnp.mean(x, axis=-1, keepdims=True)
    v = jnp.var(x, axis=-1, keepdims=True)
    return (x - m) / jnp.sqrt(v + EPS) * g + b


def _leaky(x):
    return jnp.where(x > 0, x, 0.1 * x)


def _gelu(x):
    return 0.5 * x * (1.0 + lax.erf(x * (2.0 ** -0.5)))


def _encoder_body(x_ref, w1, b1, g1, be1, w2, b2, g2, be2, o_ref):
    h = jnp.dot(x_ref[...], w1[...], preferred_element_type=jnp.float32) + b1[...]
    h = _lnorm(h, g1[...], be1[...])
    h = _leaky(h)
    h = jnp.dot(h, w2[...], preferred_element_type=jnp.float32) + b2[...]
    o_ref[...] = _lnorm(h, g2[...], be2[...])


def _layer_body(part, cntp, h_ref, skw, skb, wl, bl, wr, g, b, o_ref):
    h = h_ref[...]
    cnt = (cntp[0] + cntp[1])[:N, :1]
    cnt = jnp.maximum(cnt, 1.0)
    agg = (part[0] + part[1])[:N] / cnt
    out = (jnp.dot(agg, wl[...], preferred_element_type=jnp.float32) + bl[...]
           + jnp.dot(h, wr[...], preferred_element_type=jnp.float32))
    nrm = jnp.sqrt(jnp.sum(out * out, axis=-1, keepdims=True))
    out = out / jnp.maximum(nrm, 1e-12)
    m = jnp.mean(out, axis=0, keepdims=True)
    v = jnp.var(out, axis=0, keepdims=True)
    out = (out - m) / jnp.sqrt(v + EPS) * g[...] + b[...]
    res = jnp.dot(h, skw[...], preferred_element_type=jnp.float32) + skb[...]
    o_ref[...] = _leaky(out) + res


def _head_body(h_ref, w1, b1, g, be, w2, b2, w3, b3, o_ref):
    y = jnp.dot(h_ref[...], w1[...], preferred_element_type=jnp.float32) + b1[...]
    y = _lnorm(y, g[...], be[...])
    y = _gelu(y)
    y = jnp.dot(y, w2[...], preferred_element_type=jnp.float32) + b2[...]
    y = _gelu(y)
    y = jnp.dot(y, w3[...], preferred_element_type=jnp.float32) + b3[...]
    o_ref[...] = jax.nn.sigmoid(y)


def kernel(x, edge_index, params):
    p = params
    src = edge_index[0].astype(jnp.int32)
    dst = edge_index[1].astype(jnp.int32)
    pad = EPAD - E
    src = jnp.concatenate([src, jnp.zeros((pad,), jnp.int32)]).reshape(TG, G, CH)
    dst = jnp.concatenate([dst, jnp.full((pad,), N, jnp.int32)]).reshape(TG, G, CH)

    h = pl.pallas_call(
        _encoder_body,
        out_shape=jax.ShapeDtypeStruct((N, H), jnp.float32),
    )(x, p['enc_W1'], p['enc_b1'], p['enc_g1'], p['enc_be1'],
      p['enc_W2'], p['enc_b2'], p['enc_g2'], p['enc_be2'])

    ones_ch = jnp.ones((CH, H), jnp.float32)
    zh = jnp.zeros((NPAD, H), jnp.float32)

    cntp = _spmm_cnt(ones_ch, dst, zh).reshape(NC, NPAD, H)

    for l in range(NL):
        part = _spmm_h(h, src, dst, zh).reshape(NC, NPAD, H)
        h = pl.pallas_call(
            _layer_body,
            out_shape=jax.ShapeDtypeStruct((N, H), jnp.float32),
        )(part, cntp, h, p['skip_W'], p['skip_b'],
          p[f'sage_Wl{l}'], p[f'sage_bl{l}'], p[f'sage_Wr{l}'],
          p[f'bn_g{l}'], p[f'bn_b{l}'])

    y = pl.pallas_call(
        _head_body,
        out_shape=jax.ShapeDtypeStruct((N, DOUT), jnp.float32),
    )(h, p['head_W1'], p['head_b1'], p['head_g'], p['head_be'],
      p['head_W2'], p['head_b2'], p['head_W3'], p['head_b3'])
    return y


# speedup vs baseline: 2.8220x; 1.0004x over previous
import functools

import jax
import jax.numpy as jnp
from jax import lax
from jax.experimental import pallas as pl
from jax.experimental.pallas import tpu as pltpu
from jax.experimental.pallas import tpu_sc as plsc

N = 10000
E = 320000
H = 128
NL = 3
DOUT = 45
EPS = 1e-5

NC = 2
NS = 16
NW = NC * NS
CH = 128
G = 40
EPG = G * CH
NG0 = 3
NG1 = 1
TG = NS * (NG0 + NG1)
EPAD = TG * EPG
RPT = 632
NPAD = NS * RPT


_SC_MESH = plsc.VectorSubcoreMesh(core_axis_name="c", subcore_axis_name="s")


@functools.partial(
    pl.kernel,
    out_type=jax.ShapeDtypeStruct((NW, RPT, H), jnp.float32),
    mesh=_SC_MESH,
    scratch_types=[
        pltpu.VMEM((G, CH), jnp.int32),
        pltpu.VMEM((G, CH), jnp.int32),
        pltpu.VMEM((CH, H), jnp.float32),
        pltpu.VMEM((CH, H), jnp.float32),
        pltpu.VMEM_SHARED((NPAD, H), jnp.float32),
        pltpu.SemaphoreType.DMA,
        pltpu.SemaphoreType.DMA,
    ],
)
def _spmm_h(table_hbm, src_hbm, dst_hbm, zeros_hbm, out_hbm,
            src_v, dst_v, rows_a, rows_b, acc, sem_a, sem_b):
    c = lax.axis_index("c")
    s = lax.axis_index("s")
    wid = c * NS + s
    row0 = s * RPT
    pltpu.sync_copy(zeros_hbm.at[pl.ds(row0, RPT)], acc.at[pl.ds(row0, RPT)])
    plsc.subcore_barrier()

    def step(i, carry):
        j = 2 * i
        pltpu.make_async_copy(table_hbm.at[src_v.at[j]], rows_a, sem_a).wait()
        pltpu.async_copy(table_hbm.at[src_v.at[j + 1]], rows_b, sem_b)
        pltpu.sync_copy(rows_a, acc.at[dst_v.at[j]], add=True)
        pltpu.make_async_copy(table_hbm.at[src_v.at[j + 1]], rows_b, sem_b).wait()

        @pl.when(j + 2 < G)
        def _():
            pltpu.async_copy(table_hbm.at[src_v.at[j + 2]], rows_a, sem_a)

        pltpu.sync_copy(rows_b, acc.at[dst_v.at[j + 1]], add=True)
        return carry

    def do_groups(gbase, n):
        for g in range(n):
            pltpu.sync_copy(src_hbm.at[gbase + g], src_v)
            pltpu.sync_copy(dst_hbm.at[gbase + g], dst_v)
            pltpu.async_copy(table_hbm.at[src_v.at[0]], rows_a, sem_a)
            lax.fori_loop(0, G // 2, step, 0)

    @pl.when(c == 0)
    def _():
        do_groups(s * NG0, NG0)

    if NG1:
        @pl.when(c == 1)
        def _():
            do_groups(NS * NG0 + s, NG1)

    plsc.subcore_barrier()
    pltpu.sync_copy(acc.at[pl.ds(row0, RPT)], out_hbm.at[wid])


@functools.partial(
    pl.kernel,
    out_type=jax.ShapeDtypeStruct((NW, RPT, H), jnp.float32),
    mesh=_SC_MESH,
    scratch_types=[
        pltpu.VMEM((G, CH), jnp.int32),
        pltpu.VMEM((CH, H), jnp.float32),
        pltpu.VMEM_SHARED((NPAD, H), jnp.float32),
    ],
)
def _spmm_cnt(ones_hbm, dst_hbm, zeros_hbm, out_hbm, dst_v, ones_v, acc):
    c = lax.axis_index("c")
    s = lax.axis_index("s")
    wid = c * NS + s
    pltpu.sync_copy(ones_hbm, ones_v)
    row0 = s * RPT
    pltpu.sync_copy(zeros_hbm.at[pl.ds(row0, RPT)], acc.at[pl.ds(row0, RPT)])
    plsc.subcore_barrier()

    ngroups = TG // NW

    def step(j, carry):
        pltpu.sync_copy(ones_v, acc.at[dst_v.at[j]], add=True)
        return carry

    for g in range(ngroups):
        pltpu.sync_copy(dst_hbm.at[wid * ngroups + g], dst_v)
        lax.fori_loop(0, G, step, 0)

    plsc.subcore_barrier()
    pltpu.sync_copy(acc.at[pl.ds(row0, RPT)], out_hbm.at[wid])


def _lnorm(x, g, b):
    m = jnp.mean(x, axis=-1, keepdims=True)
    v = jnp.var(x, axis=-1, keepdims=True)
    return (x - m) / jnp.sqrt(v + EPS) * g + b


def _leaky(x):
    return jnp.where(x > 0, x, 0.1 * x)


def _gelu(x):
    return 0.5 * x * (1.0 + lax.erf(x * (2.0 ** -0.5)))


def _encoder_body(x_ref, w1, b1, g1, be1, w2, b2, g2, be2, o_ref):
    h = jnp.dot(x_ref[...], w1[...], preferred_element_type=jnp.float32) + b1[...]
    h = _lnorm(h, g1[...], be1[...])
    h = _leaky(h)
    h = jnp.dot(h, w2[...], preferred_element_type=jnp.float32) + b2[...]
    o_ref[...] = _lnorm(h, g2[...], be2[...])


def _layer_body(part, cntp, h_ref, skw, skb, wl, bl, wr, g, b, o_ref):
    h = h_ref[...]
    cnt = (cntp[0] + cntp[1])[:N, :1]
    cnt = jnp.maximum(cnt, 1.0)
    agg = (part[0] + part[1])[:N] / cnt
    out = (jnp.dot(agg, wl[...], preferred_element_type=jnp.float32) + bl[...]
           + jnp.dot(h, wr[...], preferred_element_type=jnp.float32))
    nrm = jnp.sqrt(jnp.sum(out * out, axis=-1, keepdims=True))
    out = out / jnp.maximum(nrm, 1e-12)
    m = jnp.mean(out, axis=0, keepdims=True)
    v = jnp.var(out, axis=0, keepdims=True)
    out = (out - m) / jnp.sqrt(v + EPS) * g[...] + b[...]
    res = jnp.dot(h, skw[...], preferred_element_type=jnp.float32) + skb[...]
    o_ref[...] = _leaky(out) + res


def _head_body(h_ref, w1, b1, g, be, w2, b2, w3, b3, o_ref):
    y = jnp.dot(h_ref[...], w1[...], preferred_element_type=jnp.float32) + b1[...]
    y = _lnorm(y, g[...], be[...])
    y = _gelu(y)
    y = jnp.dot(y, w2[...], preferred_element_type=jnp.float32) + b2[...]
    y = _gelu(y)
    y = jnp.dot(y, w3[...], preferred_element_type=jnp.float32) + b3[...]
    o_ref[...] = jax.nn.sigmoid(y)


def kernel(x, edge_index, params):
    p = params
    src = edge_index[0].astype(jnp.int32)
    dst = edge_index[1].astype(jnp.int32)
    pad = EPAD - E
    src = jnp.concatenate([src, jnp.zeros((pad,), jnp.int32)]).reshape(TG, G, CH)
    dst = jnp.concatenate([dst, jnp.full((pad,), N, jnp.int32)]).reshape(TG, G, CH)

    h = pl.pallas_call(
        _encoder_body,
        out_shape=jax.ShapeDtypeStruct((N, H), jnp.float32),
    )(x, p['enc_W1'], p['enc_b1'], p['enc_g1'], p['enc_be1'],
      p['enc_W2'], p['enc_b2'], p['enc_g2'], p['enc_be2'])

    ones_ch = jnp.ones((CH, H), jnp.float32)
    zh = jnp.zeros((NPAD, H), jnp.float32)

    cntp = _spmm_cnt(ones_ch, dst, zh).reshape(NC, NPAD, H)

    for l in range(NL):
        part = _spmm_h(h, src, dst, zh).reshape(NC, NPAD, H)
        h = pl.pallas_call(
            _layer_body,
            out_shape=jax.ShapeDtypeStruct((N, H), jnp.float32),
        )(part, cntp, h, p['skip_W'], p['skip_b'],
          p[f'sage_Wl{l}'], p[f'sage_bl{l}'], p[f'sage_Wr{l}'],
          p[f'bn_g{l}'], p[f'bn_b{l}'])

    y = pl.pallas_call(
        _head_body,
        out_shape=jax.ShapeDtypeStruct((N, DOUT), jnp.float32),
    )(h, p['head_W1'], p['head_b1'], p['head_g'], p['head_be'],
      p['head_W2'], p['head_b2'], p['head_W3'], p['head_b3'])
    return y
